# baseline (speedup 1.0000x reference)
import functools

import jax
import jax.numpy as jnp
from jax.experimental import pallas as pl
from jax.experimental.pallas import tpu as pltpu

P_BLK = 512


def _body(Hf, Wf, Hc, Wc, NS,
          fine_hbm, coarse_ref, ix_ref, iy_ref, x0_ref, y0_ref,
          w1f_ref, w1c_ref, b1_ref, w2t_ref, b2_ref, w3t_ref, b3_ref,
          wft_ref, bf_ref, out_ref, gbuf, dsem):
    b = pl.program_id(0)

    def issue(k, carry):
        base = k * 8
        for i in range(8):
            mi = base + i
            y0 = y0_ref[0, 0, 0, mi]
            x0 = x0_ref[0, 0, 0, mi]
            pltpu.make_async_copy(
                fine_hbm.at[b, pl.ds(y0, 2), pl.ds(x0, 2)],
                gbuf.at[mi], dsem).start()
        return carry

    jax.lax.fori_loop(0, P_BLK // 8, issue, 0)

    ix = ix_ref[0, 0]
    iy = iy_ref[0, 0]
    x0f = jnp.floor(ix)
    y0f = jnp.floor(iy)
    wx = ix - x0f
    wy = iy - y0f

    def edge_w(c0f, w, n):
        lo = jnp.where(c0f >= 0.0, 1.0 - w, 0.0)
        hi = jnp.where(c0f + 1.0 <= n - 1.0, w, 0.0)
        c0c = jnp.clip(c0f, 0.0, n - 2.0)
        w0 = jnp.where(c0c == c0f, lo, 0.0) + jnp.where(c0c == c0f + 1.0, hi, 0.0)
        c1 = c0c + 1.0
        w1 = jnp.where(c1 == c0f, lo, 0.0) + jnp.where(c1 == c0f + 1.0, hi, 0.0)
        return w0, w1

    cx0, cx1 = edge_w(x0f, wx, float(Wf))
    cy0, cy1 = edge_w(y0f, wy, float(Hf))
    c00 = cy0 * cx0
    c01 = cy0 * cx1
    c10 = cy1 * cx0
    c11 = cy1 * cx1

    iyc = (iy + 0.5) * (float(Hc) / float(Hf)) - 0.5
    ixc = (ix + 0.5) * (float(Wc) / float(Wf)) - 0.5
    yc0 = jnp.floor(iyc)
    xc0 = jnp.floor(ixc)
    wyc = iyc - yc0
    wxc = ixc - xc0
    ay0 = jnp.where(yc0 >= 0.0, 1.0 - wyc, 0.0)
    ay1 = jnp.where(yc0 + 1.0 <= Hc - 1.0, wyc, 0.0)
    bx0 = jnp.where(xc0 >= 0.0, 1.0 - wxc, 0.0)
    bx1 = jnp.where(xc0 + 1.0 <= Wc - 1.0, wxc, 0.0)
    yc0i = yc0.astype(jnp.int32)
    xc0i = xc0.astype(jnp.int32)
    ioy = jax.lax.broadcasted_iota(jnp.int32, (P_BLK, Hc), 1)
    iox = jax.lax.broadcasted_iota(jnp.int32, (P_BLK, Wc), 1)
    Ay = jnp.where(ioy == yc0i, ay0, 0.0) + jnp.where(ioy == yc0i + 1, ay1, 0.0)
    Bx = jnp.where(iox == xc0i, bx0, 0.0) + jnp.where(iox == xc0i + 1, bx1, 0.0)
    img = coarse_ref[0]
    R = jax.lax.dot_general(Ay, img, (((1,), (0,)), ((), ())),
                            preferred_element_type=jnp.float32)
    cs = jnp.sum(R * Bx, axis=1, keepdims=True)

    pltpu.make_async_copy(gbuf, gbuf, dsem).wait()

    acc = cs * w1c_ref[0:1, :]
    for s in range(NS):
        fs = (c00 * gbuf[:, 0, 0, s, :] + c01 * gbuf[:, 0, 1, s, :]
              + c10 * gbuf[:, 1, 0, s, :] + c11 * gbuf[:, 1, 1, s, :])
        acc = acc + jax.lax.dot_general(
            fs, w1f_ref[pl.ds(s * 128, 128), :], (((1,), (0,)), ((), ())),
            preferred_element_type=jnp.float32)
    h = jnp.maximum(acc + b1_ref[0:1, :], 0.0)
    h = jnp.maximum(jax.lax.dot_general(h, w2t_ref[:, :], (((1,), (0,)), ((), ())),
                                        preferred_element_type=jnp.float32)
                    + b2_ref[0:1, :], 0.0)
    h = jnp.maximum(jax.lax.dot_general(h, w3t_ref[:, :], (((1,), (0,)), ((), ())),
                                        preferred_element_type=jnp.float32)
                    + b3_ref[0:1, :], 0.0)
    out_ref[0, 0] = jax.lax.dot_general(h, wft_ref[:, :], (((1,), (0,)), ((), ())),
                                        preferred_element_type=jnp.float32) \
        + bf_ref[0, 0]


def kernel(coarse_logits, fine_features, point_coords,
           w1, b1, w2, b2, w3, b3, wf, bf):
    B, Co, Hc, Wc = coarse_logits.shape
    _, Ci, Hf, Wf = fine_features.shape
    P = point_coords.shape[1]
    NS = Ci // 128
    PB = -(-P // P_BLK)
    P_pad = PB * P_BLK

    pc = point_coords
    if P_pad != P:
        pc = jnp.pad(pc, ((0, 0), (0, P_pad - P), (0, 0)),
                     constant_values=0.5)
    ix = pc[:, :, 0] * Wf - 0.5
    iy = pc[:, :, 1] * Hf - 0.5
    x0 = jnp.clip(jnp.floor(ix), 0.0, Wf - 2.0).astype(jnp.int32)
    y0 = jnp.clip(jnp.floor(iy), 0.0, Hf - 2.0).astype(jnp.int32)

    fine_r = fine_features.transpose(0, 2, 3, 1).reshape(B, Hf, Wf, NS, 128)
    coarse = coarse_logits.reshape(B, Hc, Wc)
    ix_a = ix.reshape(B, PB, P_BLK, 1)
    iy_a = iy.reshape(B, PB, P_BLK, 1)
    x0_a = x0.reshape(B, PB, 1, P_BLK)
    y0_a = y0.reshape(B, PB, 1, P_BLK)
    fc = w1.shape[0]
    w1f = w1[:, :Ci].T
    w1c = w1[:, Ci:Ci + 1].T
    w2t = w2.T
    w3t = w3.T
    wft = wf.T
    b1r = b1.reshape(1, -1)
    b2r = b2.reshape(1, -1)
    b3r = b3.reshape(1, -1)
    bfr = bf.reshape(1, 1)

    grid = (B, PB)
    out = pl.pallas_call(
        functools.partial(_body, Hf, Wf, Hc, Wc, NS),
        grid=grid,
        in_specs=[
            pl.BlockSpec(memory_space=pl.ANY),
            pl.BlockSpec((1, Hc, Wc), lambda b, p: (b, 0, 0)),
            pl.BlockSpec((1, 1, P_BLK, 1), lambda b, p: (b, p, 0, 0)),
            pl.BlockSpec((1, 1, P_BLK, 1), lambda b, p: (b, p, 0, 0)),
            pl.BlockSpec((1, 1, 1, P_BLK), lambda b, p: (b, p, 0, 0),
                         memory_space=pltpu.SMEM),
            pl.BlockSpec((1, 1, 1, P_BLK), lambda b, p: (b, p, 0, 0),
                         memory_space=pltpu.SMEM),
            pl.BlockSpec((Ci, fc), lambda b, p: (0, 0)),
            pl.BlockSpec((1, fc), lambda b, p: (0, 0)),
            pl.BlockSpec((1, fc), lambda b, p: (0, 0)),
            pl.BlockSpec((fc, fc), lambda b, p: (0, 0)),
            pl.BlockSpec((1, fc), lambda b, p: (0, 0)),
            pl.BlockSpec((fc, fc), lambda b, p: (0, 0)),
            pl.BlockSpec((1, fc), lambda b, p: (0, 0)),
            pl.BlockSpec((fc, 1), lambda b, p: (0, 0)),
            pl.BlockSpec((1, 1), lambda b, p: (0, 0)),
        ],
        out_specs=pl.BlockSpec((1, 1, P_BLK, 1), lambda b, p: (b, p, 0, 0)),
        out_shape=jax.ShapeDtypeStruct((B, PB, P_BLK, 1), jnp.float32),
        scratch_shapes=[
            pltpu.VMEM((P_BLK, 2, 2, NS, 128), jnp.float32),
            pltpu.SemaphoreType.DMA,
        ],
        compiler_params=pltpu.CompilerParams(
            dimension_semantics=("parallel", "arbitrary")),
    )(fine_r, coarse, ix_a, iy_a, x0_a, y0_a,
      w1f, w1c, b1r, w2t, b2r, w3t, b3r, wft, bfr)

    return out.reshape(B, P_pad)[:, :P].reshape(B, 1, P)
